# baseline (speedup 1.0000x reference)
import functools

import jax
import jax.numpy as jnp
from jax import lax
from jax.experimental import pallas as pl
from jax.experimental.pallas import tpu as pltpu
from jax.experimental.pallas import tpu_sc as plsc

BB, TT, DD = 2, 2048, 1024
EE = 2
FF = 4096
NN = BB * TT
TILE = 256
NPAD = NN + TILE
NB = NPAD // TILE
NBS = NN // TILE
FT = 512
NF = FF // FT
DW = DD // 2

G = 32
GL = NN // G

NWORK = 32
SROWS = NN // NWORK
UROWS = NN // NWORK
UCHUNK = 64


@functools.cache
def _sc_mesh():
    return plsc.VectorSubcoreMesh(core_axis_name="c", subcore_axis_name="s",
                                  num_cores=2, num_subcores=16)


def _router_body(x_ref, wr_ref, b_ref, m0_ref, xb_ref):
    xblk = x_ref[...]
    logits = lax.dot_general(xblk, wr_ref[...], (((1,), (1,)), ((), ())),
                             preferred_element_type=jnp.float32)
    logits = logits + b_ref[...]
    a = logits[:, 0:1]
    b = logits[:, 1:2]
    m0_ref[...] = jnp.where(a >= b, 1.0, 0.0).astype(jnp.float32)
    xb_ref[...] = xblk.astype(jnp.bfloat16)


def _router(flat, Wr, router_bias):
    return pl.pallas_call(
        _router_body,
        grid=(1,),
        in_specs=[
            pl.BlockSpec((NN, DD), lambda i: (0, 0)),
            pl.BlockSpec((EE, DD), lambda i: (0, 0)),
            pl.BlockSpec((1, EE), lambda i: (0, 0)),
        ],
        out_specs=[
            pl.BlockSpec((NN, 1), lambda i: (0, 0)),
            pl.BlockSpec((NN, DD), lambda i: (0, 0)),
        ],
        out_shape=[
            jax.ShapeDtypeStruct((NN, 1), jnp.float32),
            jax.ShapeDtypeStruct((NN, DD), jnp.bfloat16),
        ],
    )(flat, Wr, router_bias.reshape(1, EE))


def _prefix_body(m_ref, pos_ref, meta_ref):
    m = m_ref[...]
    ii = lax.broadcasted_iota(jnp.int32, (GL, GL), 0)
    jj = lax.broadcasted_iota(jnp.int32, (GL, GL), 1)
    u = jnp.where(ii <= jj, 1.0, 0.0).astype(jnp.float32)
    p = lax.dot_general(m, u, (((1,), (0,)), ((), ())),
                        preferred_element_type=jnp.float32)
    s = jnp.sum(m, axis=1, keepdims=True)
    gi = lax.broadcasted_iota(jnp.int32, (G, G), 0)
    gj = lax.broadcasted_iota(jnp.int32, (G, G), 1)
    lt = jnp.where(gj < gi, 1.0, 0.0).astype(jnp.float32)
    off = lax.dot_general(lt, s, (((1,), (0,)), ((), ())),
                          preferred_element_type=jnp.float32)
    c0 = p + off
    gg = lax.broadcasted_iota(jnp.int32, (G, GL), 0)
    ll = lax.broadcasted_iota(jnp.int32, (G, GL), 1)
    t1 = (gg * GL + ll + 1).astype(jnp.float32)
    c1 = t1 - c0
    posf = jnp.where(m > 0.5, c0 - 1.0, NPAD - c1)
    pos_ref[...] = posf.astype(jnp.int32)

    cnt0 = jnp.sum(m).astype(jnp.int32)
    g0 = (cnt0 + (TILE - 1)) // TILE
    col = lax.broadcasted_iota(jnp.int32, (8, 128), 1)
    meta = jnp.where((col == 1) | (col == 2), g0,
                     jnp.where(col == 3, NB, 0))
    meta_ref[...] = meta.astype(jnp.int32)


def _prefix(m0):
    return pl.pallas_call(
        _prefix_body,
        grid=(1,),
        in_specs=[pl.BlockSpec((G, GL), lambda i: (0, 0))],
        out_specs=[
            pl.BlockSpec((G, GL), lambda i: (0, 0)),
            pl.BlockSpec((8, 128), lambda i: (0, 0)),
        ],
        out_shape=[
            jax.ShapeDtypeStruct((G, GL), jnp.int32),
            jax.ShapeDtypeStruct((8, 128), jnp.int32),
        ],
    )(m0)


def _scatter_body(xb32_hbm, pos_hbm, out_hbm, idx_v, rows_v, sem):
    wid = lax.axis_index("s") * 2 + lax.axis_index("c")
    base = wid * SROWS
    pltpu.sync_copy(pos_hbm.at[pl.ds(base, SROWS)], idx_v)
    pltpu.sync_copy(xb32_hbm.at[pl.ds(base, SROWS)], rows_v)
    pltpu.async_copy(rows_v, out_hbm.at[idx_v], sem).wait()


def _scatter(xb32, pos):
    return pl.kernel(
        _scatter_body,
        out_type=jax.ShapeDtypeStruct((NPAD, DW), jnp.int32),
        mesh=_sc_mesh(),
        scratch_types=[
            pltpu.VMEM((SROWS,), jnp.int32),
            pltpu.VMEM((SROWS, DW), jnp.int32),
            pltpu.SemaphoreType.DMA,
        ],
    )(xb32, pos)


def _unperm_body(ys_hbm, pos_hbm, out_hbm, idx_v, rows_v, sem):
    wid = lax.axis_index("s") * 2 + lax.axis_index("c")
    for c in range(UROWS // UCHUNK):
        base = wid * UROWS + c * UCHUNK
        pltpu.sync_copy(pos_hbm.at[pl.ds(base, UCHUNK)], idx_v)
        pltpu.async_copy(ys_hbm.at[idx_v], rows_v, sem).wait()
        pltpu.sync_copy(rows_v, out_hbm.at[pl.ds(base, UCHUNK)])


def _unperm(ys, pos):
    return pl.kernel(
        _unperm_body,
        out_type=jax.ShapeDtypeStruct((NN, DD), jnp.float32),
        mesh=_sc_mesh(),
        scratch_types=[
            pltpu.VMEM((UCHUNK,), jnp.int32),
            pltpu.VMEM((UCHUNK, DD), jnp.float32),
            pltpu.SemaphoreType.DMA,
        ],
    )(ys, pos)


def _ffn_body(meta_ref, xs_ref, wg_ref, wu_ref, wd_ref, out_ref):
    e = pl.program_id(0)
    f = pl.program_id(1)
    lo = meta_ref[2 * e]
    hi = meta_ref[2 * e + 1]
    g = wg_ref[0].astype(jnp.bfloat16)
    u = wu_ref[0].astype(jnp.bfloat16)
    d = wd_ref[0].astype(jnp.bfloat16)

    def body(t, c):
        sl = pl.ds(t * TILE, TILE)
        xblk = xs_ref[sl, :]
        gg = lax.dot_general(xblk, g, (((1,), (1,)), ((), ())),
                             preferred_element_type=jnp.float32)
        uu = lax.dot_general(xblk, u, (((1,), (1,)), ((), ())),
                             preferred_element_type=jnp.float32)
        h = (gg * jax.nn.sigmoid(gg) * uu).astype(jnp.bfloat16)
        y = lax.dot_general(h, d, (((1,), (1,)), ((), ())),
                            preferred_element_type=jnp.float32)

        @pl.when(f == 0)
        def _():
            out_ref[sl, :] = y

        @pl.when(f != 0)
        def _():
            out_ref[sl, :] = out_ref[sl, :] + y

        return c

    lax.fori_loop(lo, hi, body, 0)


def _grouped_ffn(meta, xs, Wg, Wu, Wd, n_rows, n_experts):
    grid_spec = pltpu.PrefetchScalarGridSpec(
        num_scalar_prefetch=1,
        grid=(n_experts, NF),
        in_specs=[
            pl.BlockSpec((n_rows, DD), lambda e, f, m: (0, 0)),
            pl.BlockSpec((1, FT, DD), lambda e, f, m: (e, f, 0)),
            pl.BlockSpec((1, FT, DD), lambda e, f, m: (e, f, 0)),
            pl.BlockSpec((1, DD, FT), lambda e, f, m: (e, 0, f)),
        ],
        out_specs=pl.BlockSpec((n_rows, DD), lambda e, f, m: (0, 0)),
    )
    return pl.pallas_call(
        _ffn_body,
        grid_spec=grid_spec,
        out_shape=jax.ShapeDtypeStruct((n_rows, DD), jnp.float32),
    )(meta, xs, Wg, Wu, Wd)


def _add_body(a_ref, b_ref, o_ref):
    o_ref[...] = a_ref[...] + b_ref[...]


def _add(a, b):
    return pl.pallas_call(
        _add_body,
        grid=(NBS,),
        in_specs=[
            pl.BlockSpec((TILE, DD), lambda i: (i, 0)),
            pl.BlockSpec((TILE, DD), lambda i: (i, 0)),
        ],
        out_specs=pl.BlockSpec((TILE, DD), lambda i: (i, 0)),
        out_shape=jax.ShapeDtypeStruct((NN, DD), jnp.float32),
    )(a, b)


def kernel(x, Wr, router_bias, Wg, Wu, Wd, Sg, Su, Sd):
    flat = x.reshape(NN, DD)
    m0, xb = _router(flat, Wr, router_bias)

    sh = _grouped_ffn(jnp.array([0, NBS], jnp.int32), xb,
                      Sg[None], Su[None], Sd[None], NN, 1)

    pos2d, meta8 = _prefix(m0.reshape(G, GL))
    pos = pos2d.reshape(NN)
    meta = meta8[0, :4]

    xb32 = lax.bitcast_convert_type(xb.reshape(NN, DW, 2), jnp.int32)
    xs32 = _scatter(xb32, pos)
    xs = lax.bitcast_convert_type(xs32, jnp.bfloat16).reshape(NPAD, DD)

    ys = _grouped_ffn(meta, xs, Wg, Wu, Wd, NPAD, EE)

    tmp = _unperm(ys, pos)
    out = _add(tmp, sh)
    return out.reshape(BB, TT, DD)


# speedup vs baseline: 1.1517x; 1.1517x over previous
import functools

import jax
import jax.numpy as jnp
from jax import lax
from jax.experimental import pallas as pl
from jax.experimental.pallas import tpu as pltpu
from jax.experimental.pallas import tpu_sc as plsc

BB, TT, DD = 2, 2048, 1024
EE = 2
FF = 4096
NN = BB * TT
TILE = 256
NPAD = NN + TILE
NB = NPAD // TILE
FT = 256
NF = FF // FT

G = 32
GL = NN // G

NWORK = 32
SROWS = NN // NWORK
SCHUNK = 64
UROWS = NN // NWORK
UCHUNK = 64


@functools.cache
def _sc_mesh():
    return plsc.VectorSubcoreMesh(core_axis_name="c", subcore_axis_name="s",
                                  num_cores=2, num_subcores=16)


def _router_body(x_ref, wr_ref, b_ref, m0_ref):
    logits = lax.dot_general(x_ref[...], wr_ref[...], (((1,), (1,)), ((), ())),
                             preferred_element_type=jnp.float32)
    logits = logits + b_ref[...]
    a = logits[:, 0:1]
    b = logits[:, 1:2]
    m0_ref[...] = jnp.where(a >= b, 1.0, 0.0).astype(jnp.float32)


def _router(flat, Wr, router_bias):
    return pl.pallas_call(
        _router_body,
        grid=(1,),
        in_specs=[
            pl.BlockSpec((NN, DD), lambda i: (0, 0)),
            pl.BlockSpec((EE, DD), lambda i: (0, 0)),
            pl.BlockSpec((1, EE), lambda i: (0, 0)),
        ],
        out_specs=pl.BlockSpec((NN, 1), lambda i: (0, 0)),
        out_shape=jax.ShapeDtypeStruct((NN, 1), jnp.float32),
    )(flat, Wr, router_bias.reshape(1, EE))


def _prefix_body(m_ref, pos_ref, meta_ref):
    m = m_ref[...]
    ii = lax.broadcasted_iota(jnp.int32, (GL, GL), 0)
    jj = lax.broadcasted_iota(jnp.int32, (GL, GL), 1)
    u = jnp.where(ii <= jj, 1.0, 0.0).astype(jnp.float32)
    p = lax.dot_general(m, u, (((1,), (0,)), ((), ())),
                        preferred_element_type=jnp.float32)
    s = jnp.sum(m, axis=1, keepdims=True)
    gi = lax.broadcasted_iota(jnp.int32, (G, G), 0)
    gj = lax.broadcasted_iota(jnp.int32, (G, G), 1)
    lt = jnp.where(gj < gi, 1.0, 0.0).astype(jnp.float32)
    off = lax.dot_general(lt, s, (((1,), (0,)), ((), ())),
                          preferred_element_type=jnp.float32)
    c0 = p + off
    gg = lax.broadcasted_iota(jnp.int32, (G, GL), 0)
    ll = lax.broadcasted_iota(jnp.int32, (G, GL), 1)
    t1 = (gg * GL + ll + 1).astype(jnp.float32)
    c1 = t1 - c0
    posf = jnp.where(m > 0.5, c0 - 1.0, NPAD - c1)
    pos_ref[...] = posf.astype(jnp.int32)

    cnt0 = jnp.sum(m).astype(jnp.int32)
    g0 = (cnt0 + (TILE - 1)) // TILE
    col = lax.broadcasted_iota(jnp.int32, (8, 128), 1)
    meta = jnp.where((col == 1) | (col == 2), g0,
                     jnp.where((col == 3) | (col == 5), NB, 0))
    meta_ref[...] = meta.astype(jnp.int32)


def _prefix(m0):
    return pl.pallas_call(
        _prefix_body,
        grid=(1,),
        in_specs=[pl.BlockSpec((G, GL), lambda i: (0, 0))],
        out_specs=[
            pl.BlockSpec((G, GL), lambda i: (0, 0)),
            pl.BlockSpec((8, 128), lambda i: (0, 0)),
        ],
        out_shape=[
            jax.ShapeDtypeStruct((G, GL), jnp.int32),
            jax.ShapeDtypeStruct((8, 128), jnp.int32),
        ],
    )(m0)


def _scatter_body(x_hbm, pos_hbm, out_hbm, idx_v, rows_v, sem):
    wid = lax.axis_index("s") * 2 + lax.axis_index("c")
    for c in range(SROWS // SCHUNK):
        base = wid * SROWS + c * SCHUNK
        pltpu.sync_copy(pos_hbm.at[pl.ds(base, SCHUNK)], idx_v)
        pltpu.sync_copy(x_hbm.at[pl.ds(base, SCHUNK)], rows_v)
        pltpu.async_copy(rows_v, out_hbm.at[idx_v], sem).wait()


def _scatter(flat, pos):
    return pl.kernel(
        _scatter_body,
        out_type=jax.ShapeDtypeStruct((NPAD, DD), jnp.float32),
        mesh=_sc_mesh(),
        scratch_types=[
            pltpu.VMEM((SCHUNK,), jnp.int32),
            pltpu.VMEM((SCHUNK, DD), jnp.float32),
            pltpu.SemaphoreType.DMA,
        ],
    )(flat, pos)


def _unperm_body(ys_hbm, pos_hbm, out_hbm, idx_v, rows_v, sem):
    wid = lax.axis_index("s") * 2 + lax.axis_index("c")
    for c in range(UROWS // UCHUNK):
        base = wid * UROWS + c * UCHUNK
        pltpu.sync_copy(pos_hbm.at[pl.ds(base, UCHUNK)], idx_v)
        pltpu.async_copy(ys_hbm.at[idx_v], rows_v, sem).wait()
        pltpu.sync_copy(rows_v, out_hbm.at[pl.ds(base, UCHUNK)])


def _unperm(ys, pos):
    return pl.kernel(
        _unperm_body,
        out_type=jax.ShapeDtypeStruct((NN, DD), jnp.float32),
        mesh=_sc_mesh(),
        scratch_types=[
            pltpu.VMEM((UCHUNK,), jnp.int32),
            pltpu.VMEM((UCHUNK, DD), jnp.float32),
            pltpu.SemaphoreType.DMA,
        ],
    )(ys, pos)


def _ffn_body(meta_ref, xs_ref, wg_ref, wu_ref, wd_ref,
              sg_ref, su_ref, sd_ref, out_ref):
    e = pl.program_id(0)
    f = pl.program_id(1)
    lo = meta_ref[2 * e]
    hi = meta_ref[2 * e + 1]

    def run(g, u, d, init):
        g = g.astype(jnp.bfloat16)
        u = u.astype(jnp.bfloat16)
        d = d.astype(jnp.bfloat16)

        def body(t, c):
            sl = pl.ds(t * TILE, TILE)
            xblk = xs_ref[sl, :].astype(jnp.bfloat16)
            gg = lax.dot_general(xblk, g, (((1,), (1,)), ((), ())),
                                 preferred_element_type=jnp.float32)
            uu = lax.dot_general(xblk, u, (((1,), (1,)), ((), ())),
                                 preferred_element_type=jnp.float32)
            h = (gg * jax.nn.sigmoid(gg) * uu).astype(jnp.bfloat16)
            y = lax.dot_general(h, d, (((1,), (1,)), ((), ())),
                                preferred_element_type=jnp.float32)

            @pl.when(init)
            def _():
                out_ref[sl, :] = y

            @pl.when(jnp.logical_not(init))
            def _():
                out_ref[sl, :] = out_ref[sl, :] + y

            return c

        lax.fori_loop(lo, hi, body, 0)

    @pl.when(e < EE)
    def _():
        run(wg_ref[0], wu_ref[0], wd_ref[0], f == 0)

    @pl.when(e == EE)
    def _():
        run(sg_ref[0], su_ref[0], sd_ref[0], False)


def _ffn(meta, xs, Wg, Wu, Wd, Sg, Su, Sd):
    def wmap(e, f, m):
        return (jnp.minimum(e, EE - 1), jnp.where(e == EE, 0, f), 0)

    def wmap_d(e, f, m):
        return (jnp.minimum(e, EE - 1), 0, jnp.where(e == EE, 0, f))

    def smap(e, f, m):
        return (0, jnp.where(e == EE, f, 0), 0)

    def smap_d(e, f, m):
        return (0, 0, jnp.where(e == EE, f, 0))

    grid_spec = pltpu.PrefetchScalarGridSpec(
        num_scalar_prefetch=1,
        grid=(EE + 1, NF),
        in_specs=[
            pl.BlockSpec((NPAD, DD), lambda e, f, m: (0, 0)),
            pl.BlockSpec((1, FT, DD), wmap),
            pl.BlockSpec((1, FT, DD), wmap),
            pl.BlockSpec((1, DD, FT), wmap_d),
            pl.BlockSpec((1, FT, DD), smap),
            pl.BlockSpec((1, FT, DD), smap),
            pl.BlockSpec((1, DD, FT), smap_d),
        ],
        out_specs=pl.BlockSpec((NPAD, DD), lambda e, f, m: (0, 0)),
    )
    return pl.pallas_call(
        _ffn_body,
        grid_spec=grid_spec,
        out_shape=jax.ShapeDtypeStruct((NPAD, DD), jnp.float32),
    )(meta, xs, Wg, Wu, Wd, Sg[None], Su[None], Sd[None])


def kernel(x, Wr, router_bias, Wg, Wu, Wd, Sg, Su, Sd):
    flat = x.reshape(NN, DD)
    m0 = _router(flat, Wr, router_bias)
    pos2d, meta8 = _prefix(m0.reshape(G, GL))
    pos = pos2d.reshape(NN)
    meta = meta8[0, :6]

    xs = _scatter(flat, pos)
    ys = _ffn(meta, xs, Wg, Wu, Wd, Sg, Su, Sd)
    out = _unperm(ys, pos)
    return out.reshape(BB, TT, DD)


# speedup vs baseline: 1.4538x; 1.2623x over previous
import functools

import jax
import jax.numpy as jnp
from jax import lax
from jax.experimental import pallas as pl
from jax.experimental.pallas import tpu as pltpu
from jax.experimental.pallas import tpu_sc as plsc

BB, TT, DD = 2, 2048, 1024
EE = 2
FF = 4096
NN = BB * TT
TILE = 256
NPAD = NN + TILE
NB = NPAD // TILE
FT = 512
NF = FF // FT

G = 32
GL = NN // G

NWORK = 32
SROWS = NN // NWORK
SCHUNK = 64
UROWS = NN // NWORK
UCHUNK = 64


@functools.cache
def _sc_mesh():
    return plsc.VectorSubcoreMesh(core_axis_name="c", subcore_axis_name="s",
                                  num_cores=2, num_subcores=16)


def _router_body(x_ref, wr_ref, b_ref, m0_ref):
    logits = lax.dot_general(x_ref[...], wr_ref[...], (((1,), (1,)), ((), ())),
                             preferred_element_type=jnp.float32)
    logits = logits + b_ref[...]
    a = logits[:, 0:1]
    b = logits[:, 1:2]
    m0_ref[...] = jnp.where(a >= b, 1.0, 0.0).astype(jnp.float32)


def _router(flat, Wr, router_bias):
    return pl.pallas_call(
        _router_body,
        grid=(1,),
        in_specs=[
            pl.BlockSpec((NN, DD), lambda i: (0, 0)),
            pl.BlockSpec((EE, DD), lambda i: (0, 0)),
            pl.BlockSpec((1, EE), lambda i: (0, 0)),
        ],
        out_specs=pl.BlockSpec((NN, 1), lambda i: (0, 0)),
        out_shape=jax.ShapeDtypeStruct((NN, 1), jnp.float32),
    )(flat, Wr, router_bias.reshape(1, EE))


def _prefix_body(m_ref, pos_ref, meta_ref):
    m = m_ref[...]
    ii = lax.broadcasted_iota(jnp.int32, (GL, GL), 0)
    jj = lax.broadcasted_iota(jnp.int32, (GL, GL), 1)
    u = jnp.where(ii <= jj, 1.0, 0.0).astype(jnp.float32)
    p = lax.dot_general(m, u, (((1,), (0,)), ((), ())),
                        preferred_element_type=jnp.float32)
    s = jnp.sum(m, axis=1, keepdims=True)
    gi = lax.broadcasted_iota(jnp.int32, (G, G), 0)
    gj = lax.broadcasted_iota(jnp.int32, (G, G), 1)
    lt = jnp.where(gj < gi, 1.0, 0.0).astype(jnp.float32)
    off = lax.dot_general(lt, s, (((1,), (0,)), ((), ())),
                          preferred_element_type=jnp.float32)
    c0 = p + off
    gg = lax.broadcasted_iota(jnp.int32, (G, GL), 0)
    ll = lax.broadcasted_iota(jnp.int32, (G, GL), 1)
    t1 = (gg * GL + ll + 1).astype(jnp.float32)
    c1 = t1 - c0
    posf = jnp.where(m > 0.5, c0 - 1.0, NPAD - c1)
    pos_ref[...] = posf.astype(jnp.int32)

    cnt0 = jnp.sum(m).astype(jnp.int32)
    g0 = (cnt0 + (TILE - 1)) // TILE
    col = lax.broadcasted_iota(jnp.int32, (8, 128), 1)
    meta = jnp.where((col == 1) | (col == 2), g0,
                     jnp.where((col == 3) | (col == 5), NB, 0))
    meta_ref[...] = meta.astype(jnp.int32)


def _prefix(m0):
    return pl.pallas_call(
        _prefix_body,
        grid=(1,),
        in_specs=[pl.BlockSpec((G, GL), lambda i: (0, 0))],
        out_specs=[
            pl.BlockSpec((G, GL), lambda i: (0, 0)),
            pl.BlockSpec((8, 128), lambda i: (0, 0)),
        ],
        out_shape=[
            jax.ShapeDtypeStruct((G, GL), jnp.int32),
            jax.ShapeDtypeStruct((8, 128), jnp.int32),
        ],
    )(m0)


def _scatter_body(x_hbm, pos_hbm, out_hbm, idx_v, rows_v, sem):
    wid = lax.axis_index("s") * 2 + lax.axis_index("c")
    for c in range(SROWS // SCHUNK):
        base = wid * SROWS + c * SCHUNK
        pltpu.sync_copy(pos_hbm.at[pl.ds(base, SCHUNK)], idx_v)
        pltpu.sync_copy(x_hbm.at[pl.ds(base, SCHUNK)], rows_v)
        pltpu.async_copy(rows_v, out_hbm.at[idx_v], sem).wait()


def _scatter(flat, pos):
    return pl.kernel(
        _scatter_body,
        out_type=jax.ShapeDtypeStruct((NPAD, DD), jnp.float32),
        mesh=_sc_mesh(),
        scratch_types=[
            pltpu.VMEM((SCHUNK,), jnp.int32),
            pltpu.VMEM((SCHUNK, DD), jnp.float32),
            pltpu.SemaphoreType.DMA,
        ],
    )(flat, pos)


def _unperm_body(ys_hbm, pos_hbm, out_hbm, idx_v, rows_v, sem):
    wid = lax.axis_index("s") * 2 + lax.axis_index("c")
    for c in range(UROWS // UCHUNK):
        base = wid * UROWS + c * UCHUNK
        pltpu.sync_copy(pos_hbm.at[pl.ds(base, UCHUNK)], idx_v)
        pltpu.async_copy(ys_hbm.at[idx_v], rows_v, sem).wait()
        pltpu.sync_copy(rows_v, out_hbm.at[pl.ds(base, UCHUNK)])


def _unperm(ys, pos):
    return pl.kernel(
        _unperm_body,
        out_type=jax.ShapeDtypeStruct((NN, DD), jnp.float32),
        mesh=_sc_mesh(),
        scratch_types=[
            pltpu.VMEM((UCHUNK,), jnp.int32),
            pltpu.VMEM((UCHUNK, DD), jnp.float32),
            pltpu.SemaphoreType.DMA,
        ],
    )(ys, pos)


def _cast_body(a_ref, o_ref):
    o_ref[...] = a_ref[...].astype(jnp.bfloat16)


def _cast(a):
    return pl.pallas_call(
        _cast_body,
        grid=(NB,),
        in_specs=[pl.BlockSpec((TILE, DD), lambda i: (i, 0))],
        out_specs=pl.BlockSpec((TILE, DD), lambda i: (i, 0)),
        out_shape=jax.ShapeDtypeStruct((NPAD, DD), jnp.bfloat16),
    )(a)


def _ffn_body(meta_ref, xs_ref, wg_ref, wu_ref, wd_ref,
              sg_ref, su_ref, sd_ref, out_ref):
    e = pl.program_id(0)
    f = pl.program_id(1)
    lo = meta_ref[2 * e]
    hi = meta_ref[2 * e + 1]

    def run(g, u, d, init):
        g = g.astype(jnp.bfloat16)
        u = u.astype(jnp.bfloat16)
        d = d.astype(jnp.bfloat16)

        def body(t, c):
            sl = pl.ds(t * TILE, TILE)
            xblk = xs_ref[sl, :]
            gg = lax.dot_general(xblk, g, (((1,), (1,)), ((), ())),
                                 preferred_element_type=jnp.float32)
            uu = lax.dot_general(xblk, u, (((1,), (1,)), ((), ())),
                                 preferred_element_type=jnp.float32)
            h = (gg * jax.nn.sigmoid(gg) * uu).astype(jnp.bfloat16)
            y = lax.dot_general(h, d, (((1,), (1,)), ((), ())),
                                preferred_element_type=jnp.float32)

            @pl.when(init)
            def _():
                out_ref[sl, :] = y

            @pl.when(jnp.logical_not(init))
            def _():
                out_ref[sl, :] = out_ref[sl, :] + y

            return c

        lax.fori_loop(lo, hi, body, 0)

    @pl.when(e < EE)
    def _():
        run(wg_ref[0], wu_ref[0], wd_ref[0], f == 0)

    @pl.when(e == EE)
    def _():
        run(sg_ref[0], su_ref[0], sd_ref[0], False)


def _ffn(meta, xs, Wg, Wu, Wd, Sg, Su, Sd):
    def wmap(e, f, m):
        return (jnp.minimum(e, EE - 1), jnp.where(e == EE, 0, f), 0)

    def wmap_d(e, f, m):
        return (jnp.minimum(e, EE - 1), 0, jnp.where(e == EE, 0, f))

    def smap(e, f, m):
        return (0, jnp.where(e == EE, f, 0), 0)

    def smap_d(e, f, m):
        return (0, 0, jnp.where(e == EE, f, 0))

    grid_spec = pltpu.PrefetchScalarGridSpec(
        num_scalar_prefetch=1,
        grid=(EE + 1, NF),
        in_specs=[
            pl.BlockSpec((NPAD, DD), lambda e, f, m: (0, 0)),
            pl.BlockSpec((1, FT, DD), wmap),
            pl.BlockSpec((1, FT, DD), wmap),
            pl.BlockSpec((1, DD, FT), wmap_d),
            pl.BlockSpec((1, FT, DD), smap),
            pl.BlockSpec((1, FT, DD), smap),
            pl.BlockSpec((1, DD, FT), smap_d),
        ],
        out_specs=pl.BlockSpec((NPAD, DD), lambda e, f, m: (0, 0)),
    )
    return pl.pallas_call(
        _ffn_body,
        grid_spec=grid_spec,
        out_shape=jax.ShapeDtypeStruct((NPAD, DD), jnp.float32),
    )(meta, xs, Wg, Wu, Wd, Sg[None], Su[None], Sd[None])


def kernel(x, Wr, router_bias, Wg, Wu, Wd, Sg, Su, Sd):
    flat = x.reshape(NN, DD)
    m0 = _router(flat, Wr, router_bias)
    pos2d, meta8 = _prefix(m0.reshape(G, GL))
    pos = pos2d.reshape(NN)
    meta = meta8[0, :6]

    xs = _scatter(flat, pos)
    xsb = _cast(xs)
    ys = _ffn(meta, xsb, Wg, Wu, Wd, Sg, Su, Sd)
    out = _unperm(ys, pos)
    return out.reshape(BB, TT, DD)
